# Initial kernel scaffold; baseline (speedup 1.0000x reference)
import jax
import jax.numpy as jnp
from jax.experimental import pallas as pl


def kernel(points, features, params):
    raise NotImplementedError("write your pallas kernel here")


# speedup vs baseline: 8.1059x; 8.1059x over previous
import functools
import math

import jax
import jax.numpy as jnp
from jax.experimental import pallas as pl
from jax.experimental.pallas import tpu as pltpu

_B = 256
_P = 128
_NF = 3
_LAT = 8
_K = 16
_EPS = 1e-3
_BIG = 1e30


def _leaky(x):
    return jnp.where(x >= 0.0, x, 0.1 * x)


def _mm(a, b):
    return jax.lax.dot_general(a, b, (((1,), (0,)), ((), ())),
                               preferred_element_type=jnp.float32)


def _edgeconv(pts, fts, Wd, b0, W1, b1, W2, b2, Wsc, bsc, Vb):
    ptsT = pts.T
    r = jnp.sum(pts * pts, axis=1, keepdims=True)
    rT = jnp.sum(ptsT * ptsT, axis=0, keepdims=True)
    m = _mm(pts, ptsT)
    D = r - 2.0 * m + rT

    u = _mm(fts, Wd) + b0
    lane = jax.lax.broadcasted_iota(jnp.int32, (_P, _P), 1)

    acc = jnp.zeros(u.shape, jnp.float32)
    for t in range(_K + 1):
        mn = jnp.min(D, axis=1, keepdims=True)
        eq = D <= mn
        qi = jnp.min(jnp.where(eq, lane, _P), axis=1, keepdims=True)
        onehot = lane == qi
        D = jnp.where(onehot, _BIG, D)
        if t == 0:
            continue
        g = _mm(onehot.astype(jnp.float32), Vb)
        h = _leaky(g + u)
        h = _leaky(_mm(h, W1) + b1)
        h = _leaky(_mm(h, W2) + b2)
        acc = acc + h
    fts_new = acc * (1.0 / _K)
    sc = _mm(fts, Wsc) + bsc
    return _leaky(sc + fts_new)


def _trunk_body(pts_ref, fts_ref, fs_ref, fb_ref,
                Wd0_ref, Wb0_ref, b00_ref, W01_ref, b01_ref, W02_ref, b02_ref,
                Wsc0_ref, bsc0_ref,
                Wd1_ref, Wb1_ref, b10_ref, W11_ref, b11_ref, W12_ref, b12_ref,
                Wsc1_ref, bsc1_ref,
                pool_ref):
    pts = pts_ref[0]
    fts = fs_ref[...] * fts_ref[0] + fb_ref[...]
    v0 = _mm(fts, Wb0_ref[...])
    f1 = _edgeconv(pts, fts,
                   Wd0_ref[...], b00_ref[...], W01_ref[...], b01_ref[...],
                   W02_ref[...], b02_ref[...], Wsc0_ref[...], bsc0_ref[...],
                   v0)
    v1 = _mm(f1, Wb1_ref[...])
    f2 = _edgeconv(f1, f1,
                   Wd1_ref[...], b10_ref[...], W11_ref[...], b11_ref[...],
                   W12_ref[...], b12_ref[...], Wsc1_ref[...], bsc1_ref[...],
                   v1)
    pool_ref[0] = jnp.mean(f2, axis=0, keepdims=True)


def _enc_body(pool_ref, encW_ref, encb_ref, W1_ref, b1_ref, s1_ref, t1_ref,
              z_ref, x_ref):
    z = _leaky(_mm(pool_ref[...], encW_ref[...]) + encb_ref[...])
    z_ref[...] = z
    x = _leaky(_mm(z, W1_ref[...]) + b1_ref[...])
    x_ref[...] = x * s1_ref[...] + t1_ref[...]


def _dec_body(y_ref, Wc0_ref, bc0_ref, s2_ref, t2_ref,
              Wc2_ref, bc2_ref, s3_ref, t3_ref, Wout_ref, bout_ref, out_ref):
    x = _leaky(_mm(y_ref[...], Wc0_ref[...]) + bc0_ref[...])
    x = x * s2_ref[...] + t2_ref[...]
    x = _leaky(_mm(x, Wc2_ref[...]) + bc2_ref[...])
    x = x * s3_ref[...] + t3_ref[...]
    out_ref[...] = _leaky(_mm(x, Wout_ref[...]) + bout_ref[...])


def _full_spec(shape):
    nd = len(shape)
    return pl.BlockSpec(shape, lambda i, _nd=nd: (0,) * _nd)


def kernel(points, features, params):
    p = params
    inv = 1.0 / math.sqrt(1.0 + _EPS)
    row = lambda a: a.reshape(1, -1).astype(jnp.float32)

    trunk_w = [row(p['fbn_g'] * inv), row(p['fbn_b'])]
    c_in = _NF
    for bi, ch in ((0, 32), (1, 64)):
        s0 = p[f'ec{bi}_g0'] * inv
        W0 = p[f'ec{bi}_W0'] * s0[None, :]
        Wd = W0[:c_in] - W0[c_in:]
        Wb = W0[c_in:]
        trunk_w += [Wd, Wb, row(p[f'ec{bi}_b0']),
                    p[f'ec{bi}_W1'] * (p[f'ec{bi}_g1'] * inv)[None, :],
                    row(p[f'ec{bi}_b1']),
                    p[f'ec{bi}_W2'] * (p[f'ec{bi}_g2'] * inv)[None, :],
                    row(p[f'ec{bi}_b2']),
                    p[f'ec{bi}_Wsc'] * (p[f'ec{bi}_gsc'] * inv)[None, :],
                    row(p[f'ec{bi}_bsc'])]
        c_in = ch

    pool = pl.pallas_call(
        _trunk_body,
        grid=(_B,),
        in_specs=[pl.BlockSpec((1, _P, 2), lambda i: (i, 0, 0)),
                  pl.BlockSpec((1, _P, _NF), lambda i: (i, 0, 0))]
                 + [_full_spec(w.shape) for w in trunk_w],
        out_specs=pl.BlockSpec((1, 1, 64), lambda i: (i, 0, 0)),
        out_shape=jax.ShapeDtypeStruct((_B, 1, 64), jnp.float32),
        compiler_params=pltpu.CompilerParams(
            dimension_semantics=("arbitrary",)),
    )(points, features, *trunk_w)
    pool = pool.reshape(_B, 64)

    enc_w = [p['enc_W'], row(p['enc_b']), p['dec_W1'], row(p['dec_b1']),
             row(p['dbn1_g'] * inv), row(p['dbn1_b'])]
    z, x = pl.pallas_call(
        _enc_body,
        out_shape=(jax.ShapeDtypeStruct((_B, _LAT), jnp.float32),
                   jax.ShapeDtypeStruct((_B, 25 * _P), jnp.float32)),
    )(pool, *enc_w)

    y = x.reshape(_B * _P, 25)
    dec_w = [p['dec_Wc0'], row(p['dec_bc0']), row(p['dbn2_g'] * inv),
             row(p['dbn2_b']), p['dec_Wc2'], row(p['dec_bc2']),
             row(p['dbn3_g'] * inv), row(p['dbn3_b']),
             p['dec_Wout'], row(p['dec_bout'])]
    nrows = _B * _P
    rb = 4096
    out = pl.pallas_call(
        _dec_body,
        grid=(nrows // rb,),
        in_specs=[pl.BlockSpec((rb, 25), lambda i: (i, 0))]
                 + [_full_spec(w.shape) for w in dec_w],
        out_specs=pl.BlockSpec((rb, _NF), lambda i: (i, 0)),
        out_shape=jax.ShapeDtypeStruct((nrows, _NF), jnp.float32),
        compiler_params=pltpu.CompilerParams(
            dimension_semantics=("arbitrary",)),
    )(y, *dec_w)

    return (z, out.reshape(_B, _P, _NF))


# speedup vs baseline: 17.4190x; 2.1489x over previous
import math

import jax
import jax.numpy as jnp
from jax.experimental import pallas as pl
from jax.experimental.pallas import tpu as pltpu

_B = 256
_P = 128
_NF = 3
_LAT = 8
_K = 16
_EPS = 1e-3
_BIG = 3.0e38


def _leaky(x):
    return jnp.maximum(x, 0.1 * x)


def _mm(a, b):
    return jax.lax.dot_general(a, b, (((1,), (0,)), ((), ())),
                               preferred_element_type=jnp.float32)


def _edgeconv(pts, fts, Wd, b0, W1, b1, W2, b2, Wsc, bsc, Vb):
    ptsT = pts.T
    r = jnp.sum(pts * pts, axis=1, keepdims=True)
    rT = jnp.sum(ptsT * ptsT, axis=0, keepdims=True)
    m = _mm(pts, ptsT)
    D = r - 2.0 * m + rT

    lane = jax.lax.broadcasted_iota(jnp.int32, (_P, _P), 1)
    kbits = (jax.lax.bitcast_convert_type(D + 1.0, jnp.int32) & ~127) | lane
    kf = jax.lax.bitcast_convert_type(kbits, jnp.float32)

    onehots = []
    for t in range(_K + 1):
        mn = jnp.min(kf, axis=1, keepdims=True)
        qi = jax.lax.bitcast_convert_type(mn, jnp.int32) & 127
        oh = lane == qi
        kf = jnp.where(oh, _BIG, kf)
        if t > 0:
            onehots.append(oh.astype(jnp.float32))
    S = jnp.concatenate(onehots, axis=0)

    u = _mm(fts, Wd) + b0
    ch = u.shape[-1]
    U = jnp.broadcast_to(u[None], (_K, _P, ch)).reshape(_K * _P, ch)
    H = _leaky(_mm(S, Vb) + U)
    H = _leaky(_mm(H, W1) + b1)
    H = _leaky(_mm(H, W2) + b2)
    fts_new = jnp.mean(H.reshape(_K, _P, ch), axis=0)
    sc = _mm(fts, Wsc) + bsc
    return _leaky(sc + fts_new)


def _trunk_body(pts_ref, fts_ref, fs_ref, fb_ref,
                Wd0_ref, Wb0_ref, b00_ref, W01_ref, b01_ref, W02_ref, b02_ref,
                Wsc0_ref, bsc0_ref,
                Wd1_ref, Wb1_ref, b10_ref, W11_ref, b11_ref, W12_ref, b12_ref,
                Wsc1_ref, bsc1_ref,
                pool_ref):
    pts = pts_ref[0]
    fts = fs_ref[...] * fts_ref[0] + fb_ref[...]
    v0 = _mm(fts, Wb0_ref[...])
    f1 = _edgeconv(pts, fts,
                   Wd0_ref[...], b00_ref[...], W01_ref[...], b01_ref[...],
                   W02_ref[...], b02_ref[...], Wsc0_ref[...], bsc0_ref[...],
                   v0)
    v1 = _mm(f1, Wb1_ref[...])
    f2 = _edgeconv(f1, f1,
                   Wd1_ref[...], b10_ref[...], W11_ref[...], b11_ref[...],
                   W12_ref[...], b12_ref[...], Wsc1_ref[...], bsc1_ref[...],
                   v1)
    pool_ref[0] = jnp.mean(f2, axis=0, keepdims=True)


def _enc_body(pool_ref, encW_ref, encb_ref, W1_ref, b1_ref, s1_ref, t1_ref,
              z_ref, x_ref):
    z = _leaky(_mm(pool_ref[...], encW_ref[...]) + encb_ref[...])
    z_ref[...] = z
    x = _leaky(_mm(z, W1_ref[...]) + b1_ref[...])
    x_ref[...] = x * s1_ref[...] + t1_ref[...]


def _dec_body(y_ref, Wc0_ref, bc0_ref, s2_ref, t2_ref,
              Wc2_ref, bc2_ref, s3_ref, t3_ref, Wout_ref, bout_ref, out_ref):
    x = _leaky(_mm(y_ref[...], Wc0_ref[...]) + bc0_ref[...])
    x = x * s2_ref[...] + t2_ref[...]
    x = _leaky(_mm(x, Wc2_ref[...]) + bc2_ref[...])
    x = x * s3_ref[...] + t3_ref[...]
    out_ref[...] = _leaky(_mm(x, Wout_ref[...]) + bout_ref[...])


def _full_spec(shape):
    nd = len(shape)
    return pl.BlockSpec(shape, lambda i, _nd=nd: (0,) * _nd)


def kernel(points, features, params):
    p = params
    inv = 1.0 / math.sqrt(1.0 + _EPS)
    row = lambda a: a.reshape(1, -1).astype(jnp.float32)

    trunk_w = [row(p['fbn_g'] * inv), row(p['fbn_b'])]
    c_in = _NF
    for bi, ch in ((0, 32), (1, 64)):
        s0 = p[f'ec{bi}_g0'] * inv
        W0 = p[f'ec{bi}_W0'] * s0[None, :]
        Wd = W0[:c_in] - W0[c_in:]
        Wb = W0[c_in:]
        trunk_w += [Wd, Wb, row(p[f'ec{bi}_b0']),
                    p[f'ec{bi}_W1'] * (p[f'ec{bi}_g1'] * inv)[None, :],
                    row(p[f'ec{bi}_b1']),
                    p[f'ec{bi}_W2'] * (p[f'ec{bi}_g2'] * inv)[None, :],
                    row(p[f'ec{bi}_b2']),
                    p[f'ec{bi}_Wsc'] * (p[f'ec{bi}_gsc'] * inv)[None, :],
                    row(p[f'ec{bi}_bsc'])]
        c_in = ch

    pool = pl.pallas_call(
        _trunk_body,
        grid=(_B,),
        in_specs=[pl.BlockSpec((1, _P, 2), lambda i: (i, 0, 0)),
                  pl.BlockSpec((1, _P, _NF), lambda i: (i, 0, 0))]
                 + [_full_spec(w.shape) for w in trunk_w],
        out_specs=pl.BlockSpec((1, 1, 64), lambda i: (i, 0, 0)),
        out_shape=jax.ShapeDtypeStruct((_B, 1, 64), jnp.float32),
        compiler_params=pltpu.CompilerParams(
            dimension_semantics=("arbitrary",)),
    )(points, features, *trunk_w)
    pool = pool.reshape(_B, 64)

    enc_w = [p['enc_W'], row(p['enc_b']), p['dec_W1'], row(p['dec_b1']),
             row(p['dbn1_g'] * inv), row(p['dbn1_b'])]
    z, x = pl.pallas_call(
        _enc_body,
        out_shape=(jax.ShapeDtypeStruct((_B, _LAT), jnp.float32),
                   jax.ShapeDtypeStruct((_B, 25 * _P), jnp.float32)),
    )(pool, *enc_w)

    y = x.reshape(_B * _P, 25)
    dec_w = [p['dec_Wc0'], row(p['dec_bc0']), row(p['dbn2_g'] * inv),
             row(p['dbn2_b']), p['dec_Wc2'], row(p['dec_bc2']),
             row(p['dbn3_g'] * inv), row(p['dbn3_b']),
             p['dec_Wout'], row(p['dec_bout'])]
    nrows = _B * _P
    rb = 4096
    out = pl.pallas_call(
        _dec_body,
        grid=(nrows // rb,),
        in_specs=[pl.BlockSpec((rb, 25), lambda i: (i, 0))]
                 + [_full_spec(w.shape) for w in dec_w],
        out_specs=pl.BlockSpec((rb, _NF), lambda i: (i, 0)),
        out_shape=jax.ShapeDtypeStruct((nrows, _NF), jnp.float32),
        compiler_params=pltpu.CompilerParams(
            dimension_semantics=("arbitrary",)),
    )(y, *dec_w)

    return (z, out.reshape(_B, _P, _NF))


# speedup vs baseline: 24.5143x; 1.4073x over previous
import math

import jax
import jax.numpy as jnp
from jax.experimental import pallas as pl
from jax.experimental.pallas import tpu as pltpu

_B = 256
_P = 128
_NF = 3
_LAT = 8
_K = 16
_EPS = 1e-3
_BIG = 3.0e38
_SB = 8


def _leaky(x):
    return jnp.maximum(x, 0.1 * x)


def _mm(a, b):
    return jax.lax.dot_general(a, b, (((1,), (0,)), ((), ())),
                               preferred_element_type=jnp.float32)


def _dist_stack(pts_list):
    Ds = []
    for pts in pts_list:
        r = jnp.sum(pts * pts, axis=1, keepdims=True)
        rT = jnp.broadcast_to(r, (_P, 8)).T[0:1, :]
        m = _mm(pts, pts.T)
        Ds.append(r - 2.0 * m + rT)
    return jnp.concatenate(Ds, axis=0)


def _knn_onehots(D):
    lane_f = jax.lax.broadcasted_iota(
        jnp.int32, (_SB * _P, _P), 1).astype(jnp.float32)
    ohs = []
    for t in range(_K + 1):
        mn = jnp.min(D, axis=1, keepdims=True)
        qf = jnp.min(jnp.where(D <= mn, lane_f, 1e9), axis=1, keepdims=True)
        oh = lane_f == qf
        D = jnp.where(oh, _BIG, D)
        if t > 0:
            ohs.append(oh.astype(jnp.float32))
    return ohs


def _bn(x, g, b):
    return g * x / jnp.sqrt(1.0 + _EPS) + b


def _edgeconv0(pts_list, fts_all, W0, g0, b0, W1, g1, b1, W2, g2, b2,
               Wsc, gsc, bsc):
    D = _dist_stack(pts_list)
    c = fts_all.shape[-1]
    Fcat = jnp.concatenate(
        [fts_all[s * _P:(s + 1) * _P] for s in range(_SB)], axis=1)
    Es = []
    for oh in _knn_onehots(D):
        Gbig = _mm(oh, Fcat)
        Es.append(jnp.concatenate(
            [Gbig[s * _P:(s + 1) * _P, s * c:(s + 1) * c]
             for s in range(_SB)], axis=0))
    E = jnp.concatenate(Es, axis=0)
    C = jnp.broadcast_to(fts_all[None], (_K, _SB * _P, c)).reshape(-1, c)
    x = jnp.concatenate([C, E - C], axis=1)
    x = _leaky(_bn(_mm(x, W0), g0, b0))
    x = _leaky(_bn(_mm(x, W1), g1, b1))
    x = _leaky(_bn(_mm(x, W2), g2, b2))
    ch = x.shape[-1]
    fts_new = jnp.mean(x.reshape(_K, _SB * _P, ch), axis=0)
    sc = _bn(_mm(fts_all, Wsc), gsc, bsc)
    return _leaky(sc + fts_new)


def _edgeconv(pts_list, fts_all, Wd, b0, W1, b1, W2, b2, Wsc, bsc):
    ch = b0.shape[-1]
    D = _dist_stack(pts_list)

    Vb = _mm(fts_all, Wd[1])
    npair = _SB // 2
    Vp = [jnp.concatenate([Vb[(2 * q) * _P:(2 * q + 1) * _P],
                           Vb[(2 * q + 1) * _P:(2 * q + 2) * _P]], axis=1)
          for q in range(npair)]
    u = _mm(fts_all, Wd[0]) + b0

    Es = []
    for oh in _knn_onehots(D):
        parts = []
        for q in range(npair):
            G = _mm(oh[(2 * q) * _P:(2 * q + 2) * _P], Vp[q])
            parts.append(G[:_P, :ch])
            parts.append(G[_P:, ch:])
        Es.append(jnp.concatenate(parts, axis=0))
    E = jnp.concatenate(Es, axis=0)
    U = jnp.broadcast_to(u[None], (_K, _SB * _P, ch)).reshape(-1, ch)
    H = _leaky(E + U)
    H = _leaky(_mm(H, W1) + b1)
    H = _leaky(_mm(H, W2) + b2)
    fts_new = jnp.mean(H.reshape(_K, _SB * _P, ch), axis=0)
    sc = _mm(fts_all, Wsc) + bsc
    return _leaky(sc + fts_new)


def _trunk_body(pts_ref, fts_ref, fs_ref, fb_ref, *refs):
    w0 = [r[...] for r in refs[:12]]
    w1 = [r[...] for r in refs[12:21]]
    pool_ref = refs[21]
    fts_all = _bn(fts_ref[...].reshape(_SB * _P, _NF), fs_ref[...],
                  fb_ref[...])
    f1 = _edgeconv0([pts_ref[s] for s in range(_SB)], fts_all, *w0)
    f2 = _edgeconv([f1[s * _P:(s + 1) * _P] for s in range(_SB)], f1,
                   (w1[0], w1[1]), *w1[2:])
    pool_ref[...] = jnp.mean(f2.reshape(_SB, _P, 64), axis=1, keepdims=True)


def _enc_body(pool_ref, encW_ref, encb_ref, W1_ref, b1_ref, s1_ref, t1_ref,
              z_ref, x_ref):
    z = _leaky(_mm(pool_ref[...], encW_ref[...]) + encb_ref[...])
    z_ref[...] = z
    x = _leaky(_mm(z, W1_ref[...]) + b1_ref[...])
    x_ref[...] = x * s1_ref[...] + t1_ref[...]


def _dec_body(y_ref, Wc0_ref, bc0_ref, s2_ref, t2_ref,
              Wc2_ref, bc2_ref, s3_ref, t3_ref, Wout_ref, bout_ref, out_ref):
    x = _leaky(_mm(y_ref[...], Wc0_ref[...]) + bc0_ref[...])
    x = x * s2_ref[...] + t2_ref[...]
    x = _leaky(_mm(x, Wc2_ref[...]) + bc2_ref[...])
    x = x * s3_ref[...] + t3_ref[...]
    out_ref[...] = _leaky(_mm(x, Wout_ref[...]) + bout_ref[...])


def _full_spec(shape):
    nd = len(shape)
    return pl.BlockSpec(shape, lambda i, _nd=nd: (0,) * _nd)


def kernel(points, features, params):
    p = params
    inv = 1.0 / math.sqrt(1.0 + _EPS)
    row = lambda a: a.reshape(1, -1).astype(jnp.float32)

    trunk_w = [row(p['fbn_g']), row(p['fbn_b'])]
    trunk_w += [p['ec0_W0'], row(p['ec0_g0']), row(p['ec0_b0']),
                p['ec0_W1'], row(p['ec0_g1']), row(p['ec0_b1']),
                p['ec0_W2'], row(p['ec0_g2']), row(p['ec0_b2']),
                p['ec0_Wsc'], row(p['ec0_gsc']), row(p['ec0_bsc'])]
    c_in = 32
    s0 = p['ec1_g0'] * inv
    W0 = p['ec1_W0'] * s0[None, :]
    trunk_w += [W0[:c_in] - W0[c_in:], W0[c_in:], row(p['ec1_b0']),
                p['ec1_W1'] * (p['ec1_g1'] * inv)[None, :],
                row(p['ec1_b1']),
                p['ec1_W2'] * (p['ec1_g2'] * inv)[None, :],
                row(p['ec1_b2']),
                p['ec1_Wsc'] * (p['ec1_gsc'] * inv)[None, :],
                row(p['ec1_bsc'])]

    pool = pl.pallas_call(
        _trunk_body,
        grid=(_B // _SB,),
        in_specs=[pl.BlockSpec((_SB, _P, 2), lambda i: (i, 0, 0)),
                  pl.BlockSpec((_SB, _P, _NF), lambda i: (i, 0, 0))]
                 + [_full_spec(w.shape) for w in trunk_w],
        out_specs=pl.BlockSpec((_SB, 1, 64), lambda i: (i, 0, 0)),
        out_shape=jax.ShapeDtypeStruct((_B, 1, 64), jnp.float32),
        compiler_params=pltpu.CompilerParams(
            dimension_semantics=("arbitrary",)),
    )(points, features, *trunk_w)
    pool = pool.reshape(_B, 64)

    enc_w = [p['enc_W'], row(p['enc_b']), p['dec_W1'], row(p['dec_b1']),
             row(p['dbn1_g'] * inv), row(p['dbn1_b'])]
    z, x = pl.pallas_call(
        _enc_body,
        out_shape=(jax.ShapeDtypeStruct((_B, _LAT), jnp.float32),
                   jax.ShapeDtypeStruct((_B, 25 * _P), jnp.float32)),
    )(pool, *enc_w)

    y = x.reshape(_B * _P, 25)
    dec_w = [p['dec_Wc0'], row(p['dec_bc0']), row(p['dbn2_g'] * inv),
             row(p['dbn2_b']), p['dec_Wc2'], row(p['dec_bc2']),
             row(p['dbn3_g'] * inv), row(p['dbn3_b']),
             p['dec_Wout'], row(p['dec_bout'])]
    nrows = _B * _P
    rb = 4096
    out = pl.pallas_call(
        _dec_body,
        grid=(nrows // rb,),
        in_specs=[pl.BlockSpec((rb, 25), lambda i: (i, 0))]
                 + [_full_spec(w.shape) for w in dec_w],
        out_specs=pl.BlockSpec((rb, _NF), lambda i: (i, 0)),
        out_shape=jax.ShapeDtypeStruct((nrows, _NF), jnp.float32),
        compiler_params=pltpu.CompilerParams(
            dimension_semantics=("arbitrary",)),
    )(y, *dec_w)

    return (z, out.reshape(_B, _P, _NF))


# speedup vs baseline: 26.0513x; 1.0627x over previous
import math

import jax
import jax.numpy as jnp
from jax.experimental import pallas as pl
from jax.experimental.pallas import tpu as pltpu

_B = 256
_P = 128
_NF = 3
_LAT = 8
_K = 16
_EPS = 1e-3
_BIG = 3.0e38
_SB = 8


def _leaky(x):
    return jnp.maximum(x, 0.1 * x)


def _mm(a, b):
    return jax.lax.dot_general(a, b, (((1,), (0,)), ((), ())),
                               preferred_element_type=jnp.float32)


def _dist_stack(pts_list):
    Ds = []
    for pts in pts_list:
        r = jnp.sum(pts * pts, axis=1, keepdims=True)
        rT = jnp.broadcast_to(r, (_P, 8)).T[0:1, :]
        m = _mm(pts, pts.T)
        Ds.append(r - 2.0 * m + rT)
    return jnp.concatenate(Ds, axis=0)


def _knn_onehots(D):
    lane_i = jax.lax.broadcasted_iota(jnp.int32, (_SB * _P, _P), 1)
    ohs = []
    for t in range(_K + 1):
        ai = jnp.argmin(D, axis=1, keepdims=True)
        oh = lane_i == ai
        D = jnp.where(oh, _BIG, D)
        if t > 0:
            ohs.append(oh.astype(jnp.float32))
    return ohs


def _bn(x, g, b):
    return g * x / jnp.sqrt(1.0 + _EPS) + b


def _edgeconv0(pts_list, fts_all, W0, g0, b0, W1, g1, b1, W2, g2, b2,
               Wsc, gsc, bsc):
    D = _dist_stack(pts_list)
    c = fts_all.shape[-1]
    Fcat = jnp.concatenate(
        [fts_all[s * _P:(s + 1) * _P] for s in range(_SB)], axis=1)
    Es = []
    for oh in _knn_onehots(D):
        Gbig = _mm(oh, Fcat)
        Es.append(jnp.concatenate(
            [Gbig[s * _P:(s + 1) * _P, s * c:(s + 1) * c]
             for s in range(_SB)], axis=0))
    E = jnp.concatenate(Es, axis=0)
    C = jnp.broadcast_to(fts_all[None], (_K, _SB * _P, c)).reshape(-1, c)
    x = jnp.concatenate([C, E - C], axis=1)
    x = _leaky(_bn(_mm(x, W0), g0, b0))
    x = _leaky(_bn(_mm(x, W1), g1, b1))
    x = _leaky(_bn(_mm(x, W2), g2, b2))
    ch = x.shape[-1]
    fts_new = jnp.mean(x.reshape(_K, _SB * _P, ch), axis=0)
    sc = _bn(_mm(fts_all, Wsc), gsc, bsc)
    return _leaky(sc + fts_new)


def _edgeconv(pts_list, fts_all, Wd, b0, W1, b1, W2, b2, Wsc, bsc):
    ch = b0.shape[-1]
    D = _dist_stack(pts_list)

    Vb = _mm(fts_all, Wd[1])
    npair = _SB // 2
    Vp = [jnp.concatenate([Vb[(2 * q) * _P:(2 * q + 1) * _P],
                           Vb[(2 * q + 1) * _P:(2 * q + 2) * _P]], axis=1)
          for q in range(npair)]
    u = _mm(fts_all, Wd[0]) + b0

    Es = []
    for oh in _knn_onehots(D):
        parts = []
        for q in range(npair):
            G = _mm(oh[(2 * q) * _P:(2 * q + 2) * _P], Vp[q])
            parts.append(G[:_P, :ch])
            parts.append(G[_P:, ch:])
        Es.append(jnp.concatenate(parts, axis=0))
    E = jnp.concatenate(Es, axis=0)
    U = jnp.broadcast_to(u[None], (_K, _SB * _P, ch)).reshape(-1, ch)
    H = _leaky(E + U)
    H = _leaky(_mm(H, W1) + b1)
    H = _leaky(_mm(H, W2) + b2)
    fts_new = jnp.mean(H.reshape(_K, _SB * _P, ch), axis=0)
    sc = _mm(fts_all, Wsc) + bsc
    return _leaky(sc + fts_new)


def _trunk_body(pts_ref, fts_ref, fs_ref, fb_ref, *refs):
    w0 = [r[...] for r in refs[:12]]
    w1 = [r[...] for r in refs[12:21]]
    pool_ref = refs[21]
    fts_all = _bn(fts_ref[...].reshape(_SB * _P, _NF), fs_ref[...],
                  fb_ref[...])
    f1 = _edgeconv0([pts_ref[s] for s in range(_SB)], fts_all, *w0)
    f2 = _edgeconv([f1[s * _P:(s + 1) * _P] for s in range(_SB)], f1,
                   (w1[0], w1[1]), *w1[2:])
    pool_ref[...] = jnp.mean(f2.reshape(_SB, _P, 64), axis=1, keepdims=True)


def _enc_body(pool_ref, encW_ref, encb_ref, W1_ref, b1_ref, s1_ref, t1_ref,
              z_ref, x_ref):
    z = _leaky(_mm(pool_ref[...], encW_ref[...]) + encb_ref[...])
    z_ref[...] = z
    x = _leaky(_mm(z, W1_ref[...]) + b1_ref[...])
    x_ref[...] = x * s1_ref[...] + t1_ref[...]


def _dec_body(y_ref, Wc0_ref, bc0_ref, s2_ref, t2_ref,
              Wc2_ref, bc2_ref, s3_ref, t3_ref, Wout_ref, bout_ref, out_ref):
    x = _leaky(_mm(y_ref[...], Wc0_ref[...]) + bc0_ref[...])
    x = x * s2_ref[...] + t2_ref[...]
    x = _leaky(_mm(x, Wc2_ref[...]) + bc2_ref[...])
    x = x * s3_ref[...] + t3_ref[...]
    out_ref[...] = _leaky(_mm(x, Wout_ref[...]) + bout_ref[...])


def _full_spec(shape):
    nd = len(shape)
    return pl.BlockSpec(shape, lambda i, _nd=nd: (0,) * _nd)


def kernel(points, features, params):
    p = params
    inv = 1.0 / math.sqrt(1.0 + _EPS)
    row = lambda a: a.reshape(1, -1).astype(jnp.float32)

    trunk_w = [row(p['fbn_g']), row(p['fbn_b'])]
    trunk_w += [p['ec0_W0'], row(p['ec0_g0']), row(p['ec0_b0']),
                p['ec0_W1'], row(p['ec0_g1']), row(p['ec0_b1']),
                p['ec0_W2'], row(p['ec0_g2']), row(p['ec0_b2']),
                p['ec0_Wsc'], row(p['ec0_gsc']), row(p['ec0_bsc'])]
    c_in = 32
    s0 = p['ec1_g0'] * inv
    W0 = p['ec1_W0'] * s0[None, :]
    trunk_w += [W0[:c_in] - W0[c_in:], W0[c_in:], row(p['ec1_b0']),
                p['ec1_W1'] * (p['ec1_g1'] * inv)[None, :],
                row(p['ec1_b1']),
                p['ec1_W2'] * (p['ec1_g2'] * inv)[None, :],
                row(p['ec1_b2']),
                p['ec1_Wsc'] * (p['ec1_gsc'] * inv)[None, :],
                row(p['ec1_bsc'])]

    pool = pl.pallas_call(
        _trunk_body,
        grid=(_B // _SB,),
        in_specs=[pl.BlockSpec((_SB, _P, 2), lambda i: (i, 0, 0)),
                  pl.BlockSpec((_SB, _P, _NF), lambda i: (i, 0, 0))]
                 + [_full_spec(w.shape) for w in trunk_w],
        out_specs=pl.BlockSpec((_SB, 1, 64), lambda i: (i, 0, 0)),
        out_shape=jax.ShapeDtypeStruct((_B, 1, 64), jnp.float32),
        compiler_params=pltpu.CompilerParams(
            dimension_semantics=("arbitrary",)),
    )(points, features, *trunk_w)
    pool = pool.reshape(_B, 64)

    enc_w = [p['enc_W'], row(p['enc_b']), p['dec_W1'], row(p['dec_b1']),
             row(p['dbn1_g'] * inv), row(p['dbn1_b'])]
    z, x = pl.pallas_call(
        _enc_body,
        out_shape=(jax.ShapeDtypeStruct((_B, _LAT), jnp.float32),
                   jax.ShapeDtypeStruct((_B, 25 * _P), jnp.float32)),
    )(pool, *enc_w)

    y = x.reshape(_B * _P, 25)
    dec_w = [p['dec_Wc0'], row(p['dec_bc0']), row(p['dbn2_g'] * inv),
             row(p['dbn2_b']), p['dec_Wc2'], row(p['dec_bc2']),
             row(p['dbn3_g'] * inv), row(p['dbn3_b']),
             p['dec_Wout'], row(p['dec_bout'])]
    nrows = _B * _P
    rb = 4096
    out = pl.pallas_call(
        _dec_body,
        grid=(nrows // rb,),
        in_specs=[pl.BlockSpec((rb, 25), lambda i: (i, 0))]
                 + [_full_spec(w.shape) for w in dec_w],
        out_specs=pl.BlockSpec((rb, _NF), lambda i: (i, 0)),
        out_shape=jax.ShapeDtypeStruct((nrows, _NF), jnp.float32),
        compiler_params=pltpu.CompilerParams(
            dimension_semantics=("arbitrary",)),
    )(y, *dec_w)

    return (z, out.reshape(_B, _P, _NF))
